# baseline (speedup 1.0000x reference)
import functools

import jax
import jax.numpy as jnp
from jax import lax
from jax.experimental import pallas as pl
from jax.experimental.pallas import tpu as pltpu
from jax.experimental.pallas import tpu_sc as plsc

_D = 64
_V = 1000000
_NC = 2
_NS = 16
_NW = _NC * _NS
_L = 16

_mesh = lambda: plsc.VectorSubcoreMesh(
    core_axis_name="c", subcore_axis_name="s",
    num_cores=_NC, num_subcores=_NS)

_NBLK = _V // 128
_TAIL_V = _NBLK * 128


@functools.lru_cache(maxsize=None)
def _relayout_kernel():
    n_lo = _NBLK - (_NBLK // _NW) * _NW

    @functools.partial(
        pl.kernel,
        mesh=_mesh(),
        out_type=jax.ShapeDtypeStruct((_V // 2, 128), jnp.float32),
        scratch_types=[
            pltpu.VMEM((3, _D, 128), jnp.float32),
            pltpu.VMEM((2, _D, 128), jnp.float32),
            pltpu.VMEM((_D, _D), jnp.float32),
            pltpu.VMEM((32, 128), jnp.float32),
            pltpu.SemaphoreType.DMA((3,)),
            pltpu.SemaphoreType.DMA((2,)),
            pltpu.SemaphoreType.DMA,
        ],
        compiler_params=pltpu.CompilerParams(use_tc_tiling_on_sc=True, needs_layout_passes=False),
    )
    def rel(tT_hbm, scr_hbm, buf, obuf, tbuf, tobuf, sem_i, sem_o, sem_t):
        wid = lax.axis_index("s") * _NC + lax.axis_index("c")
        start = (_NBLK // _NW) * wid + jnp.minimum(wid, n_lo)
        n = jnp.where(wid < n_lo, _NBLK // _NW + 1, _NBLK // _NW)

        def in_desc(g, make=False):
            f = pltpu.make_async_copy if make else pltpu.async_copy
            s = lax.rem(g, 3)
            vt = start + g
            return f(tT_hbm.at[:, pl.ds(vt * 128, 128)], buf.at[s],
                     sem_i.at[s])

        def out_desc(g, make=False):
            f = pltpu.make_async_copy if make else pltpu.async_copy
            s = lax.rem(g, 2)
            vt = start + g
            return f(obuf.at[s], scr_hbm.at[pl.ds(vt * 64, 64)],
                     sem_o.at[s])

        in_desc(0)
        in_desc(1)
        in_desc(2)
        iota = lax.iota(jnp.int32, _L)

        def body(i, carry):
            s = lax.rem(i, 2)
            sin = lax.rem(i, 3)
            in_desc(i, make=True).wait()

            @pl.when(i >= 2)
            def _():
                out_desc(i - 2, make=True).wait()

            rot = [lax.rem(iota + m, _L) for m in range(_L)]

            def tile(t, c2):
                c0 = (t // 2) * _L
                off = jnp.where(c0 >= _D, 1, 0)
                row_idx = iota + (c0 - _D * off)
                sto_c = iota + c0
                u0 = lax.rem(t, 2) * (2 * _L)
                for u0s in range(0, 2 * _L, _L):
                    for m in range(_L):
                        uv = u0 + u0s + rot[m]
                        col_idx = 2 * uv + off
                        v = plsc.load_gather(buf.at[sin], [row_idx, col_idx])
                        plsc.store_scatter(obuf.at[s], [uv, sto_c], v)
                return c2

            lax.fori_loop(0, 16, tile, 0)
            out_desc(i)

            @pl.when(i + 3 < n)
            def _():
                in_desc(i + 3)
            return carry

        lax.fori_loop(0, n, body, 0)
        out_desc(n - 2, make=True).wait()
        out_desc(n - 1, make=True).wait()

        @pl.when(wid == _NW - 1)
        def _():
            pltpu.async_copy(tT_hbm.at[:, pl.ds(_TAIL_V, _D)], tbuf,
                             sem_t).wait()

            rot = [lax.rem(iota + m, _L) for m in range(_L)]

            def tile_t(t, c2):
                c0 = t * _L
                off = jnp.where(c0 >= _D, 1, 0)
                row_idx = iota + (c0 - _D * off)
                sto_c = iota + c0
                for u0 in range(0, 32, _L):
                    for m in range(_L):
                        col_idx = 2 * (u0 + rot[m]) + off
                        v = plsc.load_gather(tbuf, [row_idx, col_idx])
                        plsc.store_scatter(tobuf, [u0 + rot[m], sto_c], v)
                return c2

            lax.fori_loop(0, 8, tile_t, 0)
            pltpu.async_copy(tobuf, scr_hbm.at[pl.ds(_TAIL_V // 2, 32)],
                             sem_t).wait()

    return rel


_NJ = 50
_NB = 16384
_NT = _NB // 128
_BLKS = _NJ * _NT
_BPW = _BLKS // _NW


@functools.lru_cache(maxsize=None)
def _gather_kernel():
    @functools.partial(
        pl.kernel,
        mesh=_mesh(),
        out_type=jax.ShapeDtypeStruct((_NJ, 8, 128, 8, 128), jnp.float32),
        scratch_types=[
            pltpu.VMEM((4, 128), jnp.int32),
            pltpu.VMEM((4, 128, _D), jnp.float32),
            pltpu.VMEM((4, 8, 8, 128), jnp.float32),
            pltpu.SemaphoreType.DMA((4,)),
            pltpu.SemaphoreType.DMA((4,)),
            pltpu.SemaphoreType.DMA((4,)),
        ],
        compiler_params=pltpu.CompilerParams(use_tc_tiling_on_sc=False, needs_layout_passes=False),
    )
    def gat(xT_hbm, tlin_hbm, out_hbm, idx_v, rows_v, outb, sem_i, sem_g,
            sem_o):
        wid = lax.axis_index("s") * _NC + lax.axis_index("c")
        b0 = wid * _BPW

        def idx_desc(i, make=False):
            f = pltpu.make_async_copy if make else pltpu.async_copy
            s = lax.rem(i, 4)
            blk = b0 + i
            j = blk // _NT
            bt = lax.rem(blk, _NT)
            return f(xT_hbm.at[pl.ds(j * _NB + bt * 128, 128)], idx_v.at[s],
                     sem_i.at[s])

        def gather_desc(i, make=False):
            f = pltpu.make_async_copy if make else pltpu.async_copy
            s = lax.rem(i, 4)
            return f(tlin_hbm.at[idx_v.at[s]], rows_v.at[s], sem_g.at[s])

        def out_desc(i, make=False):
            f = pltpu.make_async_copy if make else pltpu.async_copy
            s = lax.rem(i, 4)
            blk = b0 + i
            j = blk // _NT
            bt = lax.rem(blk, _NT)
            return f(outb.at[s], out_hbm.at[j, pl.ds(0, 8), bt], sem_o.at[s])

        for p in range(3):
            idx_desc(p)
        for p in range(2):
            idx_desc(p, make=True).wait()
            gather_desc(p)
        iota = lax.iota(jnp.int32, _L)

        def body(i, carry):
            s = lax.rem(i, 4)

            @pl.when(i + 2 < _BPW)
            def _():
                idx_desc(i + 2, make=True).wait()
                gather_desc(i + 2)
            gather_desc(i, make=True).wait()

            @pl.when(i >= 4)
            def _():
                out_desc(i - 4, make=True).wait()

            rot = [lax.rem(iota + m, _L) for m in range(_L)]

            def tile(t, c2):
                row_idx = iota + (t // 2) * _L
                dh0 = lax.rem(t, 2) * (2 * _L)
                for d0s in range(0, 2 * _L, _L):
                    for m in range(_L):
                        dvec = dh0 + d0s + rot[m]
                        v = plsc.load_gather(rows_v.at[s], [row_idx, dvec])
                        plsc.store_scatter(
                            outb.at[s],
                            [lax.shift_right_logical(dvec, 3),
                             lax.bitwise_and(dvec, 7), row_idx], v)
                return c2

            lax.fori_loop(0, 16, tile, 0)
            out_desc(i)

            @pl.when(i + 3 < _BPW)
            def _():
                idx_desc(i + 3)
            return carry

        lax.fori_loop(0, _BPW, body, 0)
        for g in range(_BPW - 4, _BPW):
            out_desc(g, make=True).wait()

    return gat


def kernel(x, table):
    assert x.shape == (_NB, _NJ) and table.shape == (_V, _D)
    tT = jnp.swapaxes(table, 0, 1)
    scr = _relayout_kernel()(tT)
    tlin = scr.reshape(_V, _D)
    xT = jnp.swapaxes(x, 0, 1).reshape(-1).astype(jnp.int32)
    out5 = _gather_kernel()(xT, tlin)
    return jnp.transpose(out5, (2, 4, 0, 1, 3)).reshape(_NB, _NJ, _D)


# speedup vs baseline: 1.3007x; 1.3007x over previous
import functools

import jax
import jax.numpy as jnp
from jax import lax
from jax.experimental import pallas as pl
from jax.experimental.pallas import tpu as pltpu
from jax.experimental.pallas import tpu_sc as plsc

_D = 64
_V = 1000000
_NC = 2
_NS = 16
_NW = _NC * _NS
_L = 16

_mesh = lambda: plsc.VectorSubcoreMesh(
    core_axis_name="c", subcore_axis_name="s",
    num_cores=_NC, num_subcores=_NS)

_NBLK = _V // 128
_TAIL_V = _NBLK * 128


@functools.lru_cache(maxsize=None)
def _relayout_kernel():
    n_lo = _NBLK - (_NBLK // _NW) * _NW

    @functools.partial(
        pl.kernel,
        mesh=_mesh(),
        out_type=jax.ShapeDtypeStruct((_V // 2, 128), jnp.float32),
        scratch_types=[
            pltpu.VMEM((3, _D, 128), jnp.float32),
            pltpu.VMEM((2, _D, 128), jnp.float32),
            pltpu.VMEM((_D, _D), jnp.float32),
            pltpu.VMEM((32, 128), jnp.float32),
            pltpu.SemaphoreType.DMA((3,)),
            pltpu.SemaphoreType.DMA((2,)),
            pltpu.SemaphoreType.DMA,
        ],
        compiler_params=pltpu.CompilerParams(use_tc_tiling_on_sc=True, needs_layout_passes=False),
    )
    def rel(tT_hbm, scr_hbm, buf, obuf, tbuf, tobuf, sem_i, sem_o, sem_t):
        wid = lax.axis_index("s") * _NC + lax.axis_index("c")
        start = (_NBLK // _NW) * wid + jnp.minimum(wid, n_lo)
        n = jnp.where(wid < n_lo, _NBLK // _NW + 1, _NBLK // _NW)

        def in_desc(g, make=False):
            f = pltpu.make_async_copy if make else pltpu.async_copy
            s = lax.rem(g, 3)
            vt = start + g
            return f(tT_hbm.at[:, pl.ds(vt * 128, 128)], buf.at[s],
                     sem_i.at[s])

        def out_desc(g, make=False):
            f = pltpu.make_async_copy if make else pltpu.async_copy
            s = lax.rem(g, 2)
            vt = start + g
            return f(obuf.at[s], scr_hbm.at[pl.ds(vt * 64, 64)],
                     sem_o.at[s])

        in_desc(0)
        in_desc(1)
        in_desc(2)
        iota = lax.iota(jnp.int32, _L)

        def body(i, carry):
            s = lax.rem(i, 2)
            sin = lax.rem(i, 3)
            in_desc(i, make=True).wait()

            @pl.when(i >= 2)
            def _():
                out_desc(i - 2, make=True).wait()

            rot = [lax.rem(iota + m, _L) for m in range(_L)]

            @plsc.parallel_loop(0, 16, unroll=2)
            def _(t):
                c0 = (t // 2) * _L
                off = jnp.where(c0 >= _D, 1, 0)
                row_idx = iota + (c0 - _D * off)
                sto_c = iota + c0
                u0 = lax.rem(t, 2) * (2 * _L)
                for u0s in range(0, 2 * _L, _L):
                    for m in range(_L):
                        uv = u0 + u0s + rot[m]
                        col_idx = 2 * uv + off
                        v = plsc.load_gather(buf.at[sin], [row_idx, col_idx])
                        plsc.store_scatter(obuf.at[s], [uv, sto_c], v)
            out_desc(i)

            @pl.when(i + 3 < n)
            def _():
                in_desc(i + 3)
            return carry

        lax.fori_loop(0, n, body, 0)
        out_desc(n - 2, make=True).wait()
        out_desc(n - 1, make=True).wait()

        @pl.when(wid == _NW - 1)
        def _():
            pltpu.async_copy(tT_hbm.at[:, pl.ds(_TAIL_V, _D)], tbuf,
                             sem_t).wait()

            rot = [lax.rem(iota + m, _L) for m in range(_L)]

            def tile_t(t, c2):
                c0 = t * _L
                off = jnp.where(c0 >= _D, 1, 0)
                row_idx = iota + (c0 - _D * off)
                sto_c = iota + c0
                for u0 in range(0, 32, _L):
                    for m in range(_L):
                        col_idx = 2 * (u0 + rot[m]) + off
                        v = plsc.load_gather(tbuf, [row_idx, col_idx])
                        plsc.store_scatter(tobuf, [u0 + rot[m], sto_c], v)
                return c2

            lax.fori_loop(0, 8, tile_t, 0)
            pltpu.async_copy(tobuf, scr_hbm.at[pl.ds(_TAIL_V // 2, 32)],
                             sem_t).wait()

    return rel


_NJ = 50
_NB = 16384
_NT = _NB // 128
_BLKS = _NJ * _NT
_BPW = _BLKS // _NW


@functools.lru_cache(maxsize=None)
def _gather_kernel():
    @functools.partial(
        pl.kernel,
        mesh=_mesh(),
        out_type=jax.ShapeDtypeStruct((_NJ, 8, 128, 8, 128), jnp.float32),
        scratch_types=[
            pltpu.VMEM((4, 128), jnp.int32),
            pltpu.VMEM((4, 128, _D), jnp.float32),
            pltpu.VMEM((4, 8, 8, 128), jnp.float32),
            pltpu.SemaphoreType.DMA((4,)),
            pltpu.SemaphoreType.DMA((4,)),
            pltpu.SemaphoreType.DMA((4,)),
        ],
        compiler_params=pltpu.CompilerParams(use_tc_tiling_on_sc=False, needs_layout_passes=False),
    )
    def gat(xT_hbm, tlin_hbm, out_hbm, idx_v, rows_v, outb, sem_i, sem_g,
            sem_o):
        wid = lax.axis_index("s") * _NC + lax.axis_index("c")
        b0 = wid * _BPW

        def idx_desc(i, make=False):
            f = pltpu.make_async_copy if make else pltpu.async_copy
            s = lax.rem(i, 4)
            blk = b0 + i
            j = blk // _NT
            bt = lax.rem(blk, _NT)
            return f(xT_hbm.at[pl.ds(j * _NB + bt * 128, 128)], idx_v.at[s],
                     sem_i.at[s])

        def gather_desc(i, make=False):
            f = pltpu.make_async_copy if make else pltpu.async_copy
            s = lax.rem(i, 4)
            return f(tlin_hbm.at[idx_v.at[s]], rows_v.at[s], sem_g.at[s])

        def out_desc(i, make=False):
            f = pltpu.make_async_copy if make else pltpu.async_copy
            s = lax.rem(i, 4)
            blk = b0 + i
            j = blk // _NT
            bt = lax.rem(blk, _NT)
            return f(outb.at[s], out_hbm.at[j, pl.ds(0, 8), bt], sem_o.at[s])

        for p in range(3):
            idx_desc(p)
        for p in range(2):
            idx_desc(p, make=True).wait()
            gather_desc(p)
        iota = lax.iota(jnp.int32, _L)

        def body(i, carry):
            s = lax.rem(i, 4)

            @pl.when(i + 2 < _BPW)
            def _():
                idx_desc(i + 2, make=True).wait()
                gather_desc(i + 2)
            gather_desc(i, make=True).wait()

            @pl.when(i >= 4)
            def _():
                out_desc(i - 4, make=True).wait()

            rot = [lax.rem(iota + m, _L) for m in range(_L)]

            @plsc.parallel_loop(0, 16, unroll=2)
            def _(t):
                row_idx = iota + (t // 2) * _L
                dh0 = lax.rem(t, 2) * (2 * _L)
                for d0s in range(0, 2 * _L, _L):
                    for m in range(_L):
                        dvec = dh0 + d0s + rot[m]
                        v = plsc.load_gather(rows_v.at[s], [row_idx, dvec])
                        plsc.store_scatter(
                            outb.at[s],
                            [lax.shift_right_logical(dvec, 3),
                             lax.bitwise_and(dvec, 7), row_idx], v)
            out_desc(i)

            @pl.when(i + 3 < _BPW)
            def _():
                idx_desc(i + 3)
            return carry

        lax.fori_loop(0, _BPW, body, 0)
        for g in range(_BPW - 4, _BPW):
            out_desc(g, make=True).wait()

    return gat


def kernel(x, table):
    assert x.shape == (_NB, _NJ) and table.shape == (_V, _D)
    tT = jnp.swapaxes(table, 0, 1)
    scr = _relayout_kernel()(tT)
    tlin = scr.reshape(_V, _D)
    xT = jnp.swapaxes(x, 0, 1).reshape(-1).astype(jnp.int32)
    out5 = _gather_kernel()(xT, tlin)
    return jnp.transpose(out5, (2, 4, 0, 1, 3)).reshape(_NB, _NJ, _D)
